# Initial kernel scaffold; baseline (speedup 1.0000x reference)
import jax
import jax.numpy as jnp
from jax.experimental import pallas as pl


def kernel(input_atom, input_bond, atom_graph, bond_graph, num_nbs, atom_features, W2, b2, W1, b1):
    raise NotImplementedError("write your pallas kernel here")


# speedup vs baseline: 1.0682x; 1.0682x over previous
import functools

import jax
import jax.numpy as jnp
from jax import lax
from jax.experimental import pallas as pl
from jax.experimental.pallas import tpu as pltpu
from jax.experimental.pallas import tpu_sc as plsc

B = 8
N = 6250
MAX_NB = 10
H = 128
BOND_FDIM = 5
DEPTH = 2

NPB = 6272
NP = B * NPB
ZR = N

NW = 32
NT = NP // NW
CHUNK = 8
NCHUNK = NT // CHUNK
IDX_PER_CHUNK = CHUNK * MAX_NB

TC_BLK = 512
FIN_BLK = 448
H8 = H // 16


def _row_valid(step, blk):
    rid = lax.broadcasted_iota(jnp.int32, (blk, 1), 0) + step * blk
    return lax.rem(rid, NPB) < N


def _prep_body(af_ref, bond_ref, w2a_ref, w2b_ref, ap_ref, bt_ref):
    ap_ref[...] = jnp.dot(af_ref[...], w2a_ref[...],
                          preferred_element_type=jnp.float32)
    bt_ref[...] = jnp.dot(bond_ref[...], w2b_ref[...],
                          preferred_element_type=jnp.float32)


def _update_body(af_ref, nei_ref, w1a_ref, w1b_ref, b1_ref, w2a_ref,
                 af_out_ref, ap_out_ref):
    i = pl.program_id(0)
    x = (jnp.dot(af_ref[...], w1a_ref[...], preferred_element_type=jnp.float32)
         + jnp.dot(nei_ref[...], w1b_ref[...], preferred_element_type=jnp.float32)
         + b1_ref[...])
    x = jnp.maximum(x, 0.0)
    x = jnp.where(_row_valid(i, TC_BLK), x, 0.0)
    af_out_ref[...] = x
    ap_out_ref[...] = jnp.dot(x, w2a_ref[...],
                              preferred_element_type=jnp.float32)


def _final_body(af_ref, nei_ref, w1a_ref, w1b_ref, b1_ref, out_ref):
    i = pl.program_id(0)
    x = (jnp.dot(af_ref[...], w1a_ref[...], preferred_element_type=jnp.float32)
         + jnp.dot(nei_ref[...], w1b_ref[...], preferred_element_type=jnp.float32)
         + b1_ref[...])
    x = jnp.maximum(x, 0.0)
    x = jnp.where(_row_valid(i, FIN_BLK), x, 0.0)
    part = jnp.sum(x, axis=0, keepdims=True)
    bsel = lax.broadcasted_iota(jnp.int32, (B, 1), 0) == i // (NPB // FIN_BLK)
    upd = jnp.where(bsel, part, 0.0)

    @pl.when(i == 0)
    def _():
        out_ref[...] = upd

    @pl.when(i != 0)
    def _():
        out_ref[...] += upd


def _tc_prep(af0, bond_aug, w2a, w2b_aug):
    grid = (NP // TC_BLK,)
    return pl.pallas_call(
        _prep_body,
        grid=grid,
        in_specs=[
            pl.BlockSpec((TC_BLK, H), lambda i: (i, 0)),
            pl.BlockSpec((TC_BLK, 8), lambda i: (i, 0)),
            pl.BlockSpec((H, H), lambda i: (0, 0)),
            pl.BlockSpec((8, H), lambda i: (0, 0)),
        ],
        out_specs=[
            pl.BlockSpec((TC_BLK, H), lambda i: (i, 0)),
            pl.BlockSpec((TC_BLK, H), lambda i: (i, 0)),
        ],
        out_shape=[
            jax.ShapeDtypeStruct((NP, H), jnp.float32),
            jax.ShapeDtypeStruct((NP, H), jnp.float32),
        ],
    )(af0, bond_aug, w2a, w2b_aug)


def _tc_update(af, nei, w1a, w1b, b1, w2a):
    grid = (NP // TC_BLK,)
    return pl.pallas_call(
        _update_body,
        grid=grid,
        in_specs=[
            pl.BlockSpec((TC_BLK, H), lambda i: (i, 0)),
            pl.BlockSpec((TC_BLK, H), lambda i: (i, 0)),
            pl.BlockSpec((H, H), lambda i: (0, 0)),
            pl.BlockSpec((H, H), lambda i: (0, 0)),
            pl.BlockSpec((1, H), lambda i: (0, 0)),
            pl.BlockSpec((H, H), lambda i: (0, 0)),
        ],
        out_specs=[
            pl.BlockSpec((TC_BLK, H), lambda i: (i, 0)),
            pl.BlockSpec((TC_BLK, H), lambda i: (i, 0)),
        ],
        out_shape=[
            jax.ShapeDtypeStruct((NP, H), jnp.float32),
            jax.ShapeDtypeStruct((NP, H), jnp.float32),
        ],
    )(af, nei, w1a, w1b, b1, w2a)


def _tc_final(af, nei, w1a, w1b, b1):
    grid = (NP // FIN_BLK,)
    return pl.pallas_call(
        _final_body,
        grid=grid,
        in_specs=[
            pl.BlockSpec((FIN_BLK, H), lambda i: (i, 0)),
            pl.BlockSpec((FIN_BLK, H), lambda i: (i, 0)),
            pl.BlockSpec((H, H), lambda i: (0, 0)),
            pl.BlockSpec((H, H), lambda i: (0, 0)),
            pl.BlockSpec((1, H), lambda i: (0, 0)),
        ],
        out_specs=pl.BlockSpec((B, H), lambda i: (0, 0)),
        out_shape=jax.ShapeDtypeStruct((B, H), jnp.float32),
    )(af, nei, w1a, w1b, b1)


def _sc_nei_body(atom_hbm, bond_hbm, ga_hbm, gb_hbm, out_hbm,
                 idxa_v, idxb_v, rows_a, rows_b, out_v, sem_a, sem_b):
    wid = lax.axis_index("c") * 16 + lax.axis_index("s")

    @pl.loop(0, NCHUNK)
    def _chunk(ci):
        pltpu.sync_copy(ga_hbm.at[wid, ci], idxa_v)
        pltpu.sync_copy(gb_hbm.at[wid, ci], idxb_v)
        cp_a = pltpu.async_copy(atom_hbm.at[idxa_v], rows_a, sem_a)
        cp_b = pltpu.async_copy(bond_hbm.at[idxb_v], rows_b, sem_b)
        cp_a.wait()
        cp_b.wait()

        @pl.loop(0, CHUNK)
        def _node(i):
            r0 = i * MAX_NB
            for j in range(H8):
                sl = pl.ds(j * 16, 16)
                acc = jnp.maximum(rows_a[r0, sl] + rows_b[r0, sl], 0.0)
                for k in range(1, MAX_NB):
                    acc = acc + jnp.maximum(
                        rows_a[r0 + k, sl] + rows_b[r0 + k, sl], 0.0)
                out_v[i, sl] = acc

        base = wid * NT + ci * CHUNK
        pltpu.sync_copy(out_v, out_hbm.at[pl.ds(base, CHUNK)])


def _sc_nei(atom_tab, bond_tab, ga, gb):
    mesh = plsc.VectorSubcoreMesh(core_axis_name="c", subcore_axis_name="s")
    kern = pl.kernel(
        _sc_nei_body,
        mesh=mesh,
        out_type=jax.ShapeDtypeStruct((NP, H), jnp.float32),
        scratch_types=[
            pltpu.VMEM((IDX_PER_CHUNK,), jnp.int32),
            pltpu.VMEM((IDX_PER_CHUNK,), jnp.int32),
            pltpu.VMEM((IDX_PER_CHUNK, H), jnp.float32),
            pltpu.VMEM((IDX_PER_CHUNK, H), jnp.float32),
            pltpu.VMEM((CHUNK, H), jnp.float32),
            pltpu.SemaphoreType.DMA,
            pltpu.SemaphoreType.DMA,
        ],
    )
    return kern(atom_tab, bond_tab, ga, gb)


def kernel(input_atom, input_bond, atom_graph, bond_graph, num_nbs,
           atom_features, W2, b2, W1, b1):
    del input_atom

    w2a = W2[:H]
    w2b_aug = jnp.concatenate(
        [W2[H:], b2[None, :], jnp.zeros((2, H), jnp.float32)], axis=0)
    w1a = W1[:H]
    w1b = W1[H:]
    b1r = b1[None, :]

    af0 = jnp.pad(atom_features, ((0, 0), (0, NPB - N), (0, 0)))
    af0 = af0.reshape(NP, H)
    ones = jnp.ones((B, N, 1), jnp.float32)
    bond_aug = jnp.concatenate(
        [input_bond, ones, jnp.zeros((B, N, 2), jnp.float32)], axis=2)
    bond_aug = jnp.pad(bond_aug, ((0, 0), (0, NPB - N), (0, 0)))
    bond_aug = bond_aug.reshape(NP, 8)

    valid = jnp.arange(MAX_NB, dtype=jnp.int32)[None, None, :] < num_nbs[:, :, None]
    ga = jnp.where(valid, atom_graph[..., 0] * NPB + atom_graph[..., 1], ZR)
    gb = jnp.where(valid, bond_graph[..., 0] * NPB + bond_graph[..., 1], ZR)
    ga = jnp.pad(ga, ((0, 0), (0, NPB - N), (0, 0)), constant_values=ZR)
    gb = jnp.pad(gb, ((0, 0), (0, NPB - N), (0, 0)), constant_values=ZR)
    ga = ga.reshape(NW, NCHUNK, IDX_PER_CHUNK)
    gb = gb.reshape(NW, NCHUNK, IDX_PER_CHUNK)

    atom_proj, bond_tab = _tc_prep(af0, bond_aug, w2a, w2b_aug)
    nei = _sc_nei(atom_proj, bond_tab, ga, gb)
    af1, atom_proj = _tc_update(af0, nei, w1a, w1b, b1r, w2a)
    nei = _sc_nei(atom_proj, bond_tab, ga, gb)
    return _tc_final(af1, nei, w1a, w1b, b1r)


# speedup vs baseline: 1.0688x; 1.0005x over previous
import functools

import jax
import jax.numpy as jnp
from jax import lax
from jax.experimental import pallas as pl
from jax.experimental.pallas import tpu as pltpu
from jax.experimental.pallas import tpu_sc as plsc

B = 8
N = 6250
MAX_NB = 10
H = 128
BOND_FDIM = 5
DEPTH = 2

NPB = 6272
NP = B * NPB
ZR = N

NW = 32
NT = NP // NW
CHUNK = 8
NCHUNK = NT // CHUNK
IDX_PER_CHUNK = CHUNK * MAX_NB

TC_BLK = 512
FIN_BLK = 448
H8 = H // 16


def _row_valid(step, blk):
    rid = lax.broadcasted_iota(jnp.int32, (blk, 1), 0) + step * blk
    return lax.rem(rid, NPB) < N


def _prep_body(af_ref, bond_ref, w2a_ref, w2b_ref, ap_ref, bt_ref):
    ap_ref[...] = jnp.dot(af_ref[...], w2a_ref[...],
                          preferred_element_type=jnp.float32)
    bt_ref[...] = jnp.dot(bond_ref[...], w2b_ref[...],
                          preferred_element_type=jnp.float32)


def _update_body(af_ref, nei_ref, w1a_ref, w1b_ref, b1_ref, w2a_ref,
                 af_out_ref, ap_out_ref):
    i = pl.program_id(0)
    x = (jnp.dot(af_ref[...], w1a_ref[...], preferred_element_type=jnp.float32)
         + jnp.dot(nei_ref[...], w1b_ref[...], preferred_element_type=jnp.float32)
         + b1_ref[...])
    x = jnp.maximum(x, 0.0)
    x = jnp.where(_row_valid(i, TC_BLK), x, 0.0)
    af_out_ref[...] = x
    ap_out_ref[...] = jnp.dot(x, w2a_ref[...],
                              preferred_element_type=jnp.float32)


def _final_body(af_ref, nei_ref, w1a_ref, w1b_ref, b1_ref, out_ref):
    i = pl.program_id(0)
    x = (jnp.dot(af_ref[...], w1a_ref[...], preferred_element_type=jnp.float32)
         + jnp.dot(nei_ref[...], w1b_ref[...], preferred_element_type=jnp.float32)
         + b1_ref[...])
    x = jnp.maximum(x, 0.0)
    x = jnp.where(_row_valid(i, FIN_BLK), x, 0.0)
    part = jnp.sum(x, axis=0, keepdims=True)
    bsel = lax.broadcasted_iota(jnp.int32, (B, 1), 0) == i // (NPB // FIN_BLK)
    upd = jnp.where(bsel, part, 0.0)

    @pl.when(i == 0)
    def _():
        out_ref[...] = upd

    @pl.when(i != 0)
    def _():
        out_ref[...] += upd


def _tc_prep(af0, bond_aug, w2a, w2b_aug):
    grid = (NP // TC_BLK,)
    return pl.pallas_call(
        _prep_body,
        grid=grid,
        in_specs=[
            pl.BlockSpec((TC_BLK, H), lambda i: (i, 0)),
            pl.BlockSpec((TC_BLK, 8), lambda i: (i, 0)),
            pl.BlockSpec((H, H), lambda i: (0, 0)),
            pl.BlockSpec((8, H), lambda i: (0, 0)),
        ],
        out_specs=[
            pl.BlockSpec((TC_BLK, H), lambda i: (i, 0)),
            pl.BlockSpec((TC_BLK, H), lambda i: (i, 0)),
        ],
        out_shape=[
            jax.ShapeDtypeStruct((NP, H), jnp.float32),
            jax.ShapeDtypeStruct((NP, H), jnp.float32),
        ],
    )(af0, bond_aug, w2a, w2b_aug)


def _tc_update(af, nei, w1a, w1b, b1, w2a):
    grid = (NP // TC_BLK,)
    return pl.pallas_call(
        _update_body,
        grid=grid,
        in_specs=[
            pl.BlockSpec((TC_BLK, H), lambda i: (i, 0)),
            pl.BlockSpec((TC_BLK, H), lambda i: (i, 0)),
            pl.BlockSpec((H, H), lambda i: (0, 0)),
            pl.BlockSpec((H, H), lambda i: (0, 0)),
            pl.BlockSpec((1, H), lambda i: (0, 0)),
            pl.BlockSpec((H, H), lambda i: (0, 0)),
        ],
        out_specs=[
            pl.BlockSpec((TC_BLK, H), lambda i: (i, 0)),
            pl.BlockSpec((TC_BLK, H), lambda i: (i, 0)),
        ],
        out_shape=[
            jax.ShapeDtypeStruct((NP, H), jnp.float32),
            jax.ShapeDtypeStruct((NP, H), jnp.float32),
        ],
    )(af, nei, w1a, w1b, b1, w2a)


def _tc_final(af, nei, w1a, w1b, b1):
    grid = (NP // FIN_BLK,)
    return pl.pallas_call(
        _final_body,
        grid=grid,
        in_specs=[
            pl.BlockSpec((FIN_BLK, H), lambda i: (i, 0)),
            pl.BlockSpec((FIN_BLK, H), lambda i: (i, 0)),
            pl.BlockSpec((H, H), lambda i: (0, 0)),
            pl.BlockSpec((H, H), lambda i: (0, 0)),
            pl.BlockSpec((1, H), lambda i: (0, 0)),
        ],
        out_specs=pl.BlockSpec((B, H), lambda i: (0, 0)),
        out_shape=jax.ShapeDtypeStruct((B, H), jnp.float32),
    )(af, nei, w1a, w1b, b1)


NBUF = 2
NGRP = NCHUNK // NBUF


def _sc_nei_body(atom_hbm, bond_hbm, ga_hbm, gb_hbm, out_hbm,
                 idxa_t, idxb_t, rows_a, rows_b, out_v,
                 sa0, sa1, sb0, sb1, so0, so1):
    sa = (sa0, sa1)
    sb = (sb0, sb1)
    so = (so0, so1)
    wid = lax.axis_index("c") * 16 + lax.axis_index("s")

    pltpu.sync_copy(ga_hbm.at[wid], idxa_t)
    pltpu.sync_copy(gb_hbm.at[wid], idxb_t)

    for b in range(NBUF):
        pltpu.async_copy(atom_hbm.at[idxa_t.at[b]], rows_a.at[b], sa[b])
        pltpu.async_copy(bond_hbm.at[idxb_t.at[b]], rows_b.at[b], sb[b])

    @pl.loop(0, NGRP)
    def _grp(g):
        c0 = g * NBUF
        for b in range(NBUF):
            c = c0 + b
            pltpu.make_async_copy(
                atom_hbm.at[idxa_t.at[c]], rows_a.at[b], sa[b]).wait()
            pltpu.make_async_copy(
                bond_hbm.at[idxb_t.at[c]], rows_b.at[b], sb[b]).wait()

            @pl.when(c >= NBUF)
            def _():
                prev = c - NBUF
                pltpu.make_async_copy(
                    out_v.at[b],
                    out_hbm.at[pl.ds(wid * NT + prev * CHUNK, CHUNK)],
                    so[b]).wait()

            ra = rows_a.at[b]
            rb = rows_b.at[b]
            ov = out_v.at[b]

            @pl.loop(0, CHUNK)
            def _node(i):
                r0 = i * MAX_NB
                for j in range(H8):
                    sl = pl.ds(j * 16, 16)
                    acc = jnp.maximum(ra[r0, sl] + rb[r0, sl], 0.0)
                    for k in range(1, MAX_NB):
                        acc = acc + jnp.maximum(
                            ra[r0 + k, sl] + rb[r0 + k, sl], 0.0)
                    ov[i, sl] = acc

            pltpu.async_copy(
                out_v.at[b],
                out_hbm.at[pl.ds(wid * NT + c * CHUNK, CHUNK)], so[b])

            @pl.when(c + NBUF < NCHUNK)
            def _():
                cn = c + NBUF
                pltpu.async_copy(atom_hbm.at[idxa_t.at[cn]], rows_a.at[b], sa[b])
                pltpu.async_copy(bond_hbm.at[idxb_t.at[cn]], rows_b.at[b], sb[b])

    for b in range(NBUF):
        last = NCHUNK - NBUF + b
        pltpu.make_async_copy(
            out_v.at[b],
            out_hbm.at[pl.ds(wid * NT + last * CHUNK, CHUNK)], so[b]).wait()


def _sc_nei(atom_tab, bond_tab, ga, gb):
    mesh = plsc.VectorSubcoreMesh(core_axis_name="c", subcore_axis_name="s")
    kern = pl.kernel(
        _sc_nei_body,
        mesh=mesh,
        out_type=jax.ShapeDtypeStruct((NP, H), jnp.float32),
        scratch_types=[
            pltpu.VMEM((NCHUNK, IDX_PER_CHUNK), jnp.int32),
            pltpu.VMEM((NCHUNK, IDX_PER_CHUNK), jnp.int32),
            pltpu.VMEM((NBUF, IDX_PER_CHUNK, H), jnp.float32),
            pltpu.VMEM((NBUF, IDX_PER_CHUNK, H), jnp.float32),
            pltpu.VMEM((NBUF, CHUNK, H), jnp.float32),
        ] + [pltpu.SemaphoreType.DMA] * 6,
    )
    return kern(atom_tab, bond_tab, ga, gb)


def kernel(input_atom, input_bond, atom_graph, bond_graph, num_nbs,
           atom_features, W2, b2, W1, b1):
    del input_atom

    w2a = W2[:H]
    w2b_aug = jnp.concatenate(
        [W2[H:], b2[None, :], jnp.zeros((2, H), jnp.float32)], axis=0)
    w1a = W1[:H]
    w1b = W1[H:]
    b1r = b1[None, :]

    af0 = jnp.pad(atom_features, ((0, 0), (0, NPB - N), (0, 0)))
    af0 = af0.reshape(NP, H)
    ones = jnp.ones((B, N, 1), jnp.float32)
    bond_aug = jnp.concatenate(
        [input_bond, ones, jnp.zeros((B, N, 2), jnp.float32)], axis=2)
    bond_aug = jnp.pad(bond_aug, ((0, 0), (0, NPB - N), (0, 0)))
    bond_aug = bond_aug.reshape(NP, 8)

    valid = jnp.arange(MAX_NB, dtype=jnp.int32)[None, None, :] < num_nbs[:, :, None]
    ga = jnp.where(valid, atom_graph[..., 0] * NPB + atom_graph[..., 1], ZR)
    gb = jnp.where(valid, bond_graph[..., 0] * NPB + bond_graph[..., 1], ZR)
    ga = jnp.pad(ga, ((0, 0), (0, NPB - N), (0, 0)), constant_values=ZR)
    gb = jnp.pad(gb, ((0, 0), (0, NPB - N), (0, 0)), constant_values=ZR)
    ga = ga.reshape(NW, NCHUNK, IDX_PER_CHUNK)
    gb = gb.reshape(NW, NCHUNK, IDX_PER_CHUNK)

    atom_proj, bond_tab = _tc_prep(af0, bond_aug, w2a, w2b_aug)
    nei = _sc_nei(atom_proj, bond_tab, ga, gb)
    af1, atom_proj = _tc_update(af0, nei, w1a, w1b, b1r, w2a)
    nei = _sc_nei(atom_proj, bond_tab, ga, gb)
    return _tc_final(af1, nei, w1a, w1b, b1r)


# speedup vs baseline: 21.3052x; 19.9344x over previous
import functools

import jax
import jax.numpy as jnp
from jax import lax
from jax.experimental import pallas as pl
from jax.experimental.pallas import tpu as pltpu
from jax.experimental.pallas import tpu_sc as plsc

B = 8
N = 6250
MAX_NB = 10
H = 128
BOND_FDIM = 5
DEPTH = 2

NPB = 6272
NP = B * NPB
ZR = N

NW = 32
NT = NP // NW
CHUNK = 8
NCHUNK = NT // CHUNK
IDX_PER_CHUNK = CHUNK * MAX_NB

TC_BLK = 512
FIN_BLK = 448
H32 = H // 32
W64 = H // 2


def _row_valid(step, blk):
    rid = lax.broadcasted_iota(jnp.int32, (blk, 1), 0) + step * blk
    return lax.rem(rid, NPB) < N


def _pack_bf16(x):
    lo = x[:, :W64].astype(jnp.bfloat16).astype(jnp.float32)
    hi = x[:, W64:].astype(jnp.bfloat16).astype(jnp.float32)
    lo_b = lax.bitcast_convert_type(lo, jnp.uint32) >> 16
    hi_b = lax.bitcast_convert_type(hi, jnp.uint32) & jnp.uint32(0xFFFF0000)
    return lax.bitcast_convert_type(hi_b | lo_b, jnp.int32)


def _unpack_bf16(n):
    u = lax.bitcast_convert_type(n, jnp.uint32)
    lo = lax.bitcast_convert_type(u << 16, jnp.float32)
    hi = lax.bitcast_convert_type(u & jnp.uint32(0xFFFF0000), jnp.float32)
    return lo, hi


def _prep_body(af_ref, bond_ref, w2a_ref, w2b_ref, ap_ref, bt_ref):
    ap_ref[...] = _pack_bf16(jnp.dot(af_ref[...], w2a_ref[...],
                                     preferred_element_type=jnp.float32))
    bt_ref[...] = _pack_bf16(jnp.dot(bond_ref[...], w2b_ref[...],
                                     preferred_element_type=jnp.float32))


def _update_body(af_ref, nei_ref, w1a_ref, w1b_ref, b1_ref, w2a_ref,
                 af_out_ref, ap_out_ref):
    i = pl.program_id(0)
    lo, hi = _unpack_bf16(nei_ref[...])
    x = (jnp.dot(af_ref[...], w1a_ref[...], preferred_element_type=jnp.float32)
         + jnp.dot(lo, w1b_ref[:W64], preferred_element_type=jnp.float32)
         + jnp.dot(hi, w1b_ref[W64:], preferred_element_type=jnp.float32)
         + b1_ref[...])
    x = jnp.maximum(x, 0.0)
    x = jnp.where(_row_valid(i, TC_BLK), x, 0.0)
    af_out_ref[...] = x
    ap_out_ref[...] = _pack_bf16(jnp.dot(x, w2a_ref[...],
                                         preferred_element_type=jnp.float32))


def _final_body(af_ref, nei_ref, w1a_ref, w1b_ref, b1_ref, out_ref):
    i = pl.program_id(0)
    lo, hi = _unpack_bf16(nei_ref[...])
    x = (jnp.dot(af_ref[...], w1a_ref[...], preferred_element_type=jnp.float32)
         + jnp.dot(lo, w1b_ref[:W64], preferred_element_type=jnp.float32)
         + jnp.dot(hi, w1b_ref[W64:], preferred_element_type=jnp.float32)
         + b1_ref[...])
    x = jnp.maximum(x, 0.0)
    x = jnp.where(_row_valid(i, FIN_BLK), x, 0.0)
    part = jnp.sum(x, axis=0, keepdims=True)
    bsel = lax.broadcasted_iota(jnp.int32, (B, 1), 0) == i // (NPB // FIN_BLK)
    upd = jnp.where(bsel, part, 0.0)

    @pl.when(i == 0)
    def _():
        out_ref[...] = upd

    @pl.when(i != 0)
    def _():
        out_ref[...] += upd


def _tc_prep(af0, bond_aug, w2a, w2b_aug):
    grid = (NP // TC_BLK,)
    return pl.pallas_call(
        _prep_body,
        grid=grid,
        in_specs=[
            pl.BlockSpec((TC_BLK, H), lambda i: (i, 0)),
            pl.BlockSpec((TC_BLK, 8), lambda i: (i, 0)),
            pl.BlockSpec((H, H), lambda i: (0, 0)),
            pl.BlockSpec((8, H), lambda i: (0, 0)),
        ],
        out_specs=[
            pl.BlockSpec((TC_BLK, W64), lambda i: (i, 0)),
            pl.BlockSpec((TC_BLK, W64), lambda i: (i, 0)),
        ],
        out_shape=[
            jax.ShapeDtypeStruct((NP, W64), jnp.int32),
            jax.ShapeDtypeStruct((NP, W64), jnp.int32),
        ],
    )(af0, bond_aug, w2a, w2b_aug)


def _tc_update(af, nei, w1a, w1b, b1, w2a):
    grid = (NP // TC_BLK,)
    return pl.pallas_call(
        _update_body,
        grid=grid,
        in_specs=[
            pl.BlockSpec((TC_BLK, H), lambda i: (i, 0)),
            pl.BlockSpec((TC_BLK, W64), lambda i: (i, 0)),
            pl.BlockSpec((H, H), lambda i: (0, 0)),
            pl.BlockSpec((H, H), lambda i: (0, 0)),
            pl.BlockSpec((1, H), lambda i: (0, 0)),
            pl.BlockSpec((H, H), lambda i: (0, 0)),
        ],
        out_specs=[
            pl.BlockSpec((TC_BLK, H), lambda i: (i, 0)),
            pl.BlockSpec((TC_BLK, W64), lambda i: (i, 0)),
        ],
        out_shape=[
            jax.ShapeDtypeStruct((NP, H), jnp.float32),
            jax.ShapeDtypeStruct((NP, W64), jnp.int32),
        ],
    )(af, nei, w1a, w1b, b1, w2a)


def _tc_final(af, nei, w1a, w1b, b1):
    grid = (NP // FIN_BLK,)
    return pl.pallas_call(
        _final_body,
        grid=grid,
        in_specs=[
            pl.BlockSpec((FIN_BLK, H), lambda i: (i, 0)),
            pl.BlockSpec((FIN_BLK, W64), lambda i: (i, 0)),
            pl.BlockSpec((H, H), lambda i: (0, 0)),
            pl.BlockSpec((H, H), lambda i: (0, 0)),
            pl.BlockSpec((1, H), lambda i: (0, 0)),
        ],
        out_specs=pl.BlockSpec((B, H), lambda i: (0, 0)),
        out_shape=jax.ShapeDtypeStruct((B, H), jnp.float32),
    )(af, nei, w1a, w1b, b1)


NBUF = 4
NGRP = NCHUNK // NBUF


def _sc_nei_body(atom_hbm, bond_hbm, ga_hbm, gb_hbm, out_hbm,
                 idxa_t, idxb_t, rows_a, rows_b, out_v,
                 sa0, sa1, sa2, sa3, sb0, sb1, sb2, sb3, so0, so1, so2, so3):
    sa = (sa0, sa1, sa2, sa3)
    sb = (sb0, sb1, sb2, sb3)
    so = (so0, so1, so2, so3)
    wid = lax.axis_index("c") * 16 + lax.axis_index("s")

    pltpu.sync_copy(ga_hbm.at[wid], idxa_t)
    pltpu.sync_copy(gb_hbm.at[wid], idxb_t)

    for b in range(NBUF):
        pltpu.async_copy(atom_hbm.at[idxa_t.at[b]], rows_a.at[b], sa[b])
        pltpu.async_copy(bond_hbm.at[idxb_t.at[b]], rows_b.at[b], sb[b])

    @pl.loop(0, NGRP)
    def _grp(g):
        c0 = g * NBUF
        for b in range(NBUF):
            c = c0 + b
            pltpu.make_async_copy(
                atom_hbm.at[idxa_t.at[c]], rows_a.at[b], sa[b]).wait()
            pltpu.make_async_copy(
                bond_hbm.at[idxb_t.at[c]], rows_b.at[b], sb[b]).wait()

            @pl.when(c >= NBUF)
            def _():
                prev = c - NBUF
                pltpu.make_async_copy(
                    out_v.at[b],
                    out_hbm.at[pl.ds(wid * NT + prev * CHUNK, CHUNK)],
                    so[b]).wait()

            ra = rows_a.at[b]
            rb = rows_b.at[b]
            ov = out_v.at[b]
            zero = jnp.zeros((), jnp.bfloat16)

            @pl.loop(0, CHUNK)
            def _node(i):
                r0 = i * MAX_NB
                for j in range(H32):
                    sl = pl.ds(j * 16, 16)
                    def bfp(ref, r):
                        return plsc.bitcast(ref[r, sl], jnp.bfloat16)
                    acc = jnp.maximum(bfp(ra, r0) + bfp(rb, r0), zero)
                    for k in range(1, MAX_NB):
                        acc = acc + jnp.maximum(
                            bfp(ra, r0 + k) + bfp(rb, r0 + k), zero)
                    ov[i, sl] = plsc.bitcast(acc, jnp.int32)

            pltpu.async_copy(
                out_v.at[b],
                out_hbm.at[pl.ds(wid * NT + c * CHUNK, CHUNK)], so[b])

            @pl.when(c + NBUF < NCHUNK)
            def _():
                cn = c + NBUF
                pltpu.async_copy(atom_hbm.at[idxa_t.at[cn]], rows_a.at[b], sa[b])
                pltpu.async_copy(bond_hbm.at[idxb_t.at[cn]], rows_b.at[b], sb[b])

    for b in range(NBUF):
        last = NCHUNK - NBUF + b
        pltpu.make_async_copy(
            out_v.at[b],
            out_hbm.at[pl.ds(wid * NT + last * CHUNK, CHUNK)], so[b]).wait()


def _sc_nei(atom_tab, bond_tab, ga, gb):
    mesh = plsc.VectorSubcoreMesh(core_axis_name="c", subcore_axis_name="s")
    kern = pl.kernel(
        _sc_nei_body,
        mesh=mesh,
        out_type=jax.ShapeDtypeStruct((NP, W64), jnp.int32),
        scratch_types=[
            pltpu.VMEM((NCHUNK, IDX_PER_CHUNK), jnp.int32),
            pltpu.VMEM((NCHUNK, IDX_PER_CHUNK), jnp.int32),
            pltpu.VMEM((NBUF, IDX_PER_CHUNK, W64), jnp.int32),
            pltpu.VMEM((NBUF, IDX_PER_CHUNK, W64), jnp.int32),
            pltpu.VMEM((NBUF, CHUNK, W64), jnp.int32),
        ] + [pltpu.SemaphoreType.DMA] * 12,
        compiler_params=pltpu.CompilerParams(needs_layout_passes=False,
                                             use_tc_tiling_on_sc=False),
    )
    return kern(atom_tab, bond_tab, ga, gb)


def kernel(input_atom, input_bond, atom_graph, bond_graph, num_nbs,
           atom_features, W2, b2, W1, b1):
    del input_atom

    w2a = W2[:H]
    w2b_aug = jnp.concatenate(
        [W2[H:], b2[None, :], jnp.zeros((2, H), jnp.float32)], axis=0)
    w1a = W1[:H]
    w1b = W1[H:]
    b1r = b1[None, :]

    af0 = jnp.pad(atom_features, ((0, 0), (0, NPB - N), (0, 0)))
    af0 = af0.reshape(NP, H)
    ones = jnp.ones((B, N, 1), jnp.float32)
    bond_aug = jnp.concatenate(
        [input_bond, ones, jnp.zeros((B, N, 2), jnp.float32)], axis=2)
    bond_aug = jnp.pad(bond_aug, ((0, 0), (0, NPB - N), (0, 0)))
    bond_aug = bond_aug.reshape(NP, 8)

    valid = jnp.arange(MAX_NB, dtype=jnp.int32)[None, None, :] < num_nbs[:, :, None]
    slot = (jnp.arange(N, dtype=jnp.int32)[None, :, None] * MAX_NB
            + jnp.arange(MAX_NB, dtype=jnp.int32)[None, None, :])
    bbase = jnp.arange(B, dtype=jnp.int32)[:, None, None] * NPB
    zrow = bbase + N + slot % (NPB - N)
    ga = jnp.where(valid, atom_graph[..., 0] * NPB + atom_graph[..., 1], zrow)
    gb = jnp.where(valid, bond_graph[..., 0] * NPB + bond_graph[..., 1], zrow)
    padslot = (jnp.arange(NPB - N, dtype=jnp.int32)[None, :, None] * MAX_NB
               + jnp.arange(MAX_NB, dtype=jnp.int32)[None, None, :])
    padrow = jnp.broadcast_to(bbase + N + padslot % (NPB - N),
                              (B, NPB - N, MAX_NB))
    ga = jnp.concatenate([ga, padrow], axis=1)
    gb = jnp.concatenate([gb, padrow], axis=1)
    ga = ga.reshape(NW, NCHUNK, IDX_PER_CHUNK)
    gb = gb.reshape(NW, NCHUNK, IDX_PER_CHUNK)

    atom_proj, bond_tab = _tc_prep(af0, bond_aug, w2a, w2b_aug)
    nei = _sc_nei(atom_proj, bond_tab, ga, gb)
    af1, atom_proj = _tc_update(af0, nei, w1a, w1b, b1r, w2a)
    nei = _sc_nei(atom_proj, bond_tab, ga, gb)
    return _tc_final(af1, nei, w1a, w1b, b1r)
